# baseline (speedup 1.0000x reference)
import functools

import jax
import jax.numpy as jnp
from jax import lax
from jax.experimental import pallas as pl
from jax.experimental.pallas import tpu as pltpu

B, T, DM = 1, 2048, 2048
H, DK, DV, P = 6, 256, 256, 4
CHUNK = 64
NCHUNK = T // CHUNK
WINDOW = 1024
GATE_NORM = 16.0
EPS = 1e-5

F32 = jnp.float32
BF16 = jnp.bfloat16


def _dot(a, b):
    return lax.dot_general(a, b, (((1,), (0,)), ((), ())),
                           preferred_element_type=F32)


def _dot_t(a, b):
    return lax.dot_general(a, b, (((1,), (1,)), ((), ())),
                           preferred_element_type=F32)


def _dot_0(a, b):
    return lax.dot_general(a, b, (((0,), (0,)), ((), ())),
                           preferred_element_type=F32)


def _log_sigmoid(x):
    return jnp.minimum(x, 0.0) - jnp.log1p(jnp.exp(-jnp.abs(x)))


def _compiler_params(sem, vmem_mb):
    return pltpu.CompilerParams(dimension_semantics=sem,
                                vmem_limit_bytes=vmem_mb * 1024 * 1024)


def _sse_proj_body(xb_ref, wq_ref, wk_ref, wv_ref, wlq0_ref, wlq1_ref,
                   wlk0_ref, wlk1_ref, wgk0_ref, wgk1_ref, bgk_ref,
                   wg0_ref, wg1_ref, q_out, k_out, v_out, g_out, gate_out):
    xb = xb_ref[...]
    q_raw = _dot(xb, wq_ref[...])
    mq = _dot(_dot(xb, wlq0_ref[...]).astype(BF16), wlq1_ref[...])
    q_out[...] = (q_raw * mq).astype(BF16)
    k_raw = _dot(xb, wk_ref[...])
    mk = _dot(_dot(xb, wlk0_ref[...]).astype(BF16), wlk1_ref[...])
    k = k_raw * mk
    parts = []
    for h in range(H):
        kh = k[:, h * DK:(h + 1) * DK]
        m = jnp.max(kh, axis=-1, keepdims=True)
        e = jnp.exp(kh - m)
        parts.append(e / jnp.sum(e, axis=-1, keepdims=True))
    k_out[...] = jnp.concatenate(parts, axis=-1).astype(BF16)
    v_out[...] = _dot(xb, wv_ref[...]).astype(BF16)
    g_pre = _dot(_dot(xb, wgk0_ref[...]).astype(BF16), wgk1_ref[...]) \
        + bgk_ref[...]
    g_out[...] = _log_sigmoid(g_pre) / GATE_NORM
    gt = _dot(_dot(xb, wg0_ref[...]).astype(BF16), wg1_ref[...])
    gate_out[...] = gt * (1.0 / (1.0 + jnp.exp(-gt)))


def _sse_proj(xb, wq, wk, wv, wlq0, wlq1, wlk0, wlk1, wgk0p, wgk1p, bgk,
              wg0, wg1):
    bt = 512
    grid = (T // bt,)
    row = lambda i: (i, 0)
    fixed = lambda i: (0, 0)
    wspec = lambda w: pl.BlockSpec(w.shape, fixed)
    bf_sd = jax.ShapeDtypeStruct((T, H * DK), BF16)
    f32_sd = jax.ShapeDtypeStruct((T, H * DK), F32)
    return pl.pallas_call(
        _sse_proj_body,
        grid=grid,
        in_specs=[pl.BlockSpec((bt, DM), row)] + [wspec(w) for w in (
            wq, wk, wv, wlq0, wlq1, wlk0, wlk1, wgk0p, wgk1p, bgk, wg0, wg1)],
        out_specs=[pl.BlockSpec((bt, H * DK), row)] * 5,
        out_shape=[bf_sd, bf_sd, bf_sd, f32_sd, f32_sd],
        compiler_params=_compiler_params(("arbitrary",), 56),
        name="sse_proj",
    )(xb, wq, wk, wv, wlq0, wlq1, wlk0, wlk1, wgk0p, wgk1p, bgk, wg0, wg1)


def _swa_proj_body(xb_ref, wsq_ref, wsk_ref, wsv_ref, cos_ref, sin_ref,
                   q_out, k_out, v_out):
    xb = xb_ref[...]
    c = cos_ref[...]
    s = sin_ref[...]

    def rope(raw):
        parts = []
        for h in range(H):
            x1 = raw[:, h * DK:h * DK + DK // 2]
            x2 = raw[:, h * DK + DK // 2:(h + 1) * DK]
            parts.append(x1 * c - x2 * s)
            parts.append(x2 * c + x1 * s)
        return jnp.concatenate(parts, axis=-1).astype(BF16)

    q_out[...] = rope(_dot(xb, wsq_ref[...]))
    k_out[...] = rope(_dot(xb, wsk_ref[...]))
    v_out[...] = _dot(xb, wsv_ref[...]).astype(BF16)


def _swa_proj(xb, wsq, wsk, wsv, cos, sin):
    bt = 1024
    grid = (T // bt,)
    row = lambda i: (i, 0)
    fixed = lambda i: (0, 0)
    wspec = lambda w: pl.BlockSpec(w.shape, fixed)
    out_sd = jax.ShapeDtypeStruct((T, H * DK), BF16)
    return pl.pallas_call(
        _swa_proj_body,
        grid=grid,
        in_specs=[pl.BlockSpec((bt, DM), row), wspec(wsq), wspec(wsk),
                  wspec(wsv), pl.BlockSpec((bt, DK // 2), row),
                  pl.BlockSpec((bt, DK // 2), row)],
        out_specs=[pl.BlockSpec((bt, H * DK), row)] * 3,
        out_shape=[out_sd] * 3,
        compiler_params=_compiler_params(("arbitrary",), 52),
        name="swa_proj",
    )(xb, wsq, wsk, wsv, cos, sin)


GCH = 128


GH = 2


def _gla_body(q_ref, k_ref, v_ref, g_ref, roh_ref, rrep_ref, o_ref, s_ref):
    s_ref[...] = jnp.zeros_like(s_ref)
    ri = lax.broadcasted_iota(jnp.int32, (GCH, GCH), 0)
    ci = lax.broadcasted_iota(jnp.int32, (GCH, GCH), 1)
    ltri = jnp.where(ri >= ci, 1.0, 0.0).astype(F32)

    def chunk(c, carry):
        sl = pl.ds(pl.multiple_of(c * GCH, GCH), GCH)
        rc = roh_ref[sl, :]
        mm = _dot_t(rc, rc) * ltri
        rr = rrep_ref[sl, :]
        for j in range(GH):
            hs = slice(j * DK, (j + 1) * DK)
            gc = g_ref[sl, hs]
            G = _dot(ltri, gc)
            Gt = G[GCH - 1:GCH, :]
            kc = k_ref[sl, hs]
            qib = (q_ref[sl, hs] * jnp.exp(G)).astype(BF16)
            ke = (kc * jnp.exp(-G)).astype(BF16)
            kd = (kc * jnp.exp(Gt - G)).astype(BF16)
            A = _dot_t(qib, ke) * mm
            vc = v_ref[sl, hs]
            oc = _dot(A.astype(BF16), vc)
            qi_st = jnp.concatenate([qib] * P, axis=1) * rr
            kd_st = jnp.concatenate([kd] * P, axis=1) * rr
            egt_rep = jnp.concatenate([jnp.exp(Gt)] * P, axis=1)
            sj = s_ref[pl.ds(j * DV, DV), :]
            oc += _dot_t(qi_st, sj.astype(BF16))
            s_ref[pl.ds(j * DV, DV), :] = sj * egt_rep + _dot_0(vc, kd_st)
            o_ref[sl, hs] = oc
        return carry

    lax.fori_loop(0, T // GCH, chunk, 0)


def _gla(q, k, v, g, r_oh, r_rep):
    grid = (H // GH,)
    head = lambda h: (0, h)
    fixed = lambda h: (0, 0)
    return pl.pallas_call(
        _gla_body,
        grid=grid,
        in_specs=[pl.BlockSpec((T, GH * DK), head)] * 3 + [
            pl.BlockSpec((T, GH * DK), head),
            pl.BlockSpec((T, 128), fixed),
            pl.BlockSpec((T, P * DK), fixed)],
        out_specs=pl.BlockSpec((T, GH * DV), head),
        out_shape=jax.ShapeDtypeStruct((T, H * DV), F32),
        scratch_shapes=[pltpu.VMEM((GH * DV, P * DK), F32)],
        compiler_params=_compiler_params(("arbitrary",), 48),
        name="sse_gla",
    )(q, k, v, g, r_oh, r_rep)


def _swa_body(q_ref, kp_ref, vp_ref, o_ref):
    bq = q_ref.shape[0]
    span = bq + WINDOW
    qb = pl.program_id(1)
    sl = pl.ds(pl.multiple_of(qb * bq, bq), span)
    ksl = kp_ref[sl, :]
    s = _dot_t(q_ref[...], ksl) * (DK ** -0.5)
    tl = lax.broadcasted_iota(jnp.int32, (bq, span), 0)
    jj = lax.broadcasted_iota(jnp.int32, (bq, span), 1)
    allowed = (jj > tl) & (jj <= tl + WINDOW) & (jj + qb * bq >= WINDOW)
    s = jnp.where(allowed, s, -1e30)
    m = jnp.max(s, axis=-1, keepdims=True)
    e = jnp.exp(s - m)
    denom = jnp.sum(e, axis=-1, keepdims=True)
    oc = _dot(e.astype(BF16), vp_ref[sl, :])
    o_ref[...] = (oc / denom).astype(BF16)


def _swa(qs, ks_pad, vs_pad):
    bq = 1024
    grid = (H, T // bq)
    qmap = lambda h, i: (i, h)
    kvmap = lambda h, i: (0, h)
    return pl.pallas_call(
        _swa_body,
        grid=grid,
        in_specs=[pl.BlockSpec((bq, DK), qmap),
                  pl.BlockSpec((T + WINDOW, DK), kvmap),
                  pl.BlockSpec((T + WINDOW, DK), kvmap)],
        out_specs=pl.BlockSpec((bq, DK), qmap),
        out_shape=jax.ShapeDtypeStruct((T, H * DK), BF16),
        compiler_params=_compiler_params(("arbitrary", "arbitrary"), 50),
        name="swa_attn",
    )(qs, ks_pad, vs_pad)


def _rms(x, w):
    return x * lax.rsqrt(jnp.mean(x * x, axis=-1, keepdims=True) + EPS) * w


def _merge_body(o_ref, gate_ref, swa_ref, wso_ref, wwo_ref, wnorm_ref,
                wm1_ref, wm2_ref, out_ref):
    o = o_ref[...]
    wn = wnorm_ref[...]
    parts = []
    for h in range(H):
        parts.append(_rms(o[:, h * DV:(h + 1) * DV], wn))
    gated = (jnp.concatenate(parts, axis=-1) * gate_ref[...]).astype(BF16)
    s1 = _dot(gated, wso_ref[...])
    s2 = _dot(swa_ref[...], wwo_ref[...])
    out_ref[...] = _rms(s1, wm1_ref[...]) + _rms(s2, wm2_ref[...])


def _merge(o_sse, gate, swa, wso, wwo, wnorm, wm1, wm2):
    bt = 512
    grid = (T // bt,)
    row = lambda i: (i, 0)
    fixed = lambda i: (0, 0)
    wspec = lambda w: pl.BlockSpec(w.shape, fixed)
    return pl.pallas_call(
        _merge_body,
        grid=grid,
        in_specs=[pl.BlockSpec((bt, H * DV), row),
                  pl.BlockSpec((bt, H * DV), row),
                  pl.BlockSpec((bt, H * DK), row),
                  wspec(wso), wspec(wwo), wspec(wnorm), wspec(wm1),
                  wspec(wm2)],
        out_specs=pl.BlockSpec((bt, DM), row),
        out_shape=jax.ShapeDtypeStruct((T, DM), F32),
        compiler_params=_compiler_params(("arbitrary",), 48),
        name="merge_out",
    )(o_sse, gate, swa, wso, wwo, wnorm, wm1, wm2)


def kernel(x, params):
    p = params
    x2 = x[0]
    xb = x2.astype(BF16)

    e = x @ p['W_e']
    vals, idx = lax.top_k(e, 1)
    ws = jax.nn.softmax(vals, axis=-1)
    oh = jax.nn.one_hot(idx, P, dtype=e.dtype)
    w_route = jnp.einsum('btkp,btk->btp', oh, ws)[0]
    r_oh = jnp.pad(w_route, ((0, 0), (0, 128 - P))).astype(BF16)
    r_rep = jnp.repeat(w_route, DK, axis=1).astype(BF16)

    inv = 10000.0 ** (-jnp.arange(0, DK, 2, dtype=F32) / DK)
    fr = jnp.arange(T, dtype=F32)[:, None] * inv[None, :]
    cos, sin = jnp.cos(fr), jnp.sin(fr)

    wq = p['W_sse_q'].astype(BF16)
    wk = p['W_sse_k'].astype(BF16)
    wv = p['W_sse_v'].astype(BF16)
    wlq0 = p['W_lq0'].astype(BF16)
    wlq1 = p['W_lq1'].astype(BF16)
    wlk0 = p['W_lk0'].astype(BF16)
    wlk1 = p['W_lk1'].astype(BF16)
    wgk0p = jnp.pad(p['W_gk0'], ((0, 0), (0, 112))).astype(BF16)
    wgk1p = jnp.pad(p['W_gk1'], ((0, 112), (0, 0))).astype(BF16)
    bgk = p['b_gk1'].reshape(1, -1)
    wg0 = p['W_g0'].astype(BF16)
    wg1 = p['W_g1'].astype(BF16)
    wsq = p['W_swa_q'].astype(BF16)
    wsk = p['W_swa_k'].astype(BF16)
    wsv = p['W_swa_v'].astype(BF16)
    wso = p['W_sse_o'].astype(BF16)
    wwo = p['W_swa_o'].astype(BF16)

    q, k, v, g, gate = _sse_proj(xb, wq, wk, wv, wlq0, wlq1, wlk0, wlk1,
                                 wgk0p, wgk1p, bgk, wg0, wg1)
    qs, ks, vs = _swa_proj(xb, wsq, wsk, wsv, cos, sin)
    o_sse = _gla(q, k, v, g, r_oh, r_rep)
    ks_pad = jnp.pad(ks, ((WINDOW, 0), (0, 0)))
    vs_pad = jnp.pad(vs, ((WINDOW, 0), (0, 0)))
    swa = _swa(qs, ks_pad, vs_pad)
    out = _merge(o_sse, gate, swa, wso, wwo,
                 p['w_onorm'].reshape(1, -1),
                 p['w_sse_merge'].reshape(1, -1),
                 p['w_swa_merge'].reshape(1, -1))
    return out.reshape(B, T, DM)


# speedup vs baseline: 1.1522x; 1.1522x over previous
import jax
import jax.numpy as jnp
from jax import lax
from jax.experimental import pallas as pl
from jax.experimental.pallas import tpu as pltpu

B, T, DM = 1, 2048, 2048
H, DK, DV, P = 6, 256, 256, 4
WINDOW = 1024
GATE_NORM = 16.0
EPS = 1e-5

F32 = jnp.float32
BF16 = jnp.bfloat16


def _dot(a, b):
    return lax.dot_general(a, b, (((1,), (0,)), ((), ())),
                           preferred_element_type=F32)


def _dot_t(a, b):
    return lax.dot_general(a, b, (((1,), (1,)), ((), ())),
                           preferred_element_type=F32)


def _dot_0(a, b):
    return lax.dot_general(a, b, (((0,), (0,)), ((), ())),
                           preferred_element_type=F32)


def _log_sigmoid(x):
    return jnp.minimum(x, 0.0) - jnp.log1p(jnp.exp(-jnp.abs(x)))


def _compiler_params(sem, vmem_mb):
    return pltpu.CompilerParams(dimension_semantics=sem,
                                vmem_limit_bytes=vmem_mb * 1024 * 1024)


CB = 256
RS = 512


def _stream_mats(mats, wbuf, wbf, sems, consume):
    ncb = H * DK // CB
    nst = len(mats) * ncb

    def dma(stage):
        m, cb = divmod(stage, ncb)
        return pltpu.make_async_copy(
            mats[m].at[:, pl.ds(cb * CB, CB)], wbuf.at[stage % 2],
            sems.at[stage % 2])

    dma(0).start()
    dma(1).start()
    for stage in range(nst):
        m, cb = divmod(stage, ncb)
        dma(stage).wait()
        wbf[...] = wbuf[stage % 2].astype(BF16)
        if stage + 2 < nst:
            dma(stage + 2).start()
        consume(m, cb)


def _mega_sse_body(xb_ref, wq_hbm, wk_hbm, wv_hbm, smw_ref, wlq1_ref,
                   wlk1_ref, wgk1_ref, wg1_ref, bgk_ref,
                   q_out, k_out, v_out, g_out, gate_out, wbuf, wbf, sm, sems):
    for rs in range(T // RS):
        r = pl.ds(rs * RS, RS)
        sm[r, :] = _dot(xb_ref[r, :], smw_ref[...]).astype(BF16)

    def consume(m, cb):
        cbs = pl.ds(cb * CB, CB)
        for rs in range(T // RS):
            r = pl.ds(rs * RS, RS)
            raw = _dot(xb_ref[r, :], wbf[...])
            if m == 0:
                mq = _dot(sm[r, 0:256], wlq1_ref[:, cbs])
                q_out[r, cbs] = (raw * mq).astype(BF16)
            elif m == 1:
                mk = _dot(sm[r, 256:512], wlk1_ref[:, cbs])
                kblk = raw * mk
                parts = []
                for gi in range(CB // DK):
                    kh = kblk[:, gi * DK:(gi + 1) * DK]
                    mx = jnp.max(kh, axis=-1, keepdims=True)
                    ee = jnp.exp(kh - mx)
                    parts.append(ee / jnp.sum(ee, axis=-1, keepdims=True))
                k_out[r, cbs] = jnp.concatenate(parts, axis=-1).astype(BF16)
            else:
                v_out[r, cbs] = raw.astype(BF16)

    _stream_mats([wq_hbm, wk_hbm, wv_hbm], wbuf, wbf, sems, consume)
    for rs in range(T // RS):
        r = pl.ds(rs * RS, RS)
        gp = _dot(sm[r, 768:896], wgk1_ref[...]) + bgk_ref[...]
        g_out[r, :] = (_log_sigmoid(gp) / GATE_NORM).astype(BF16)
        gt = _dot(sm[r, 512:768], wg1_ref[...])
        gate_out[r, :] = (gt * (1.0 / (1.0 + jnp.exp(-gt)))).astype(BF16)


def _sse_proj(xb, wq, wk, wv, smw, wlq1, wlk1, wgk1p, wg1, bgk):
    vm = lambda: pl.BlockSpec(memory_space=pltpu.VMEM)
    hbm = lambda: pl.BlockSpec(memory_space=pl.ANY)
    bf_sd = jax.ShapeDtypeStruct((T, H * DK), BF16)
    return pl.pallas_call(
        _mega_sse_body,
        in_specs=[vm(), hbm(), hbm(), hbm(), vm(), vm(), vm(), vm(), vm(),
                  vm()],
        out_specs=[vm()] * 5,
        out_shape=[bf_sd] * 5,
        scratch_shapes=[pltpu.VMEM((2, DM, CB), F32),
                        pltpu.VMEM((DM, CB), BF16),
                        pltpu.VMEM((T, 896), BF16),
                        pltpu.SemaphoreType.DMA((2,))],
        compiler_params=_compiler_params((), 56),
        name="sse_proj",
    )(xb, wq, wk, wv, smw, wlq1, wlk1, wgk1p, wg1, bgk)


def _mega_swa_body(xb_ref, wsq_hbm, wsk_hbm, wsv_hbm, cos_ref, sin_ref,
                   q_out, k_out, v_out, wbuf, wbf, sems):
    outs = [q_out, k_out, v_out]

    def consume(m, cb):
        cbs = pl.ds(cb * CB, CB)
        for rs in range(T // RS):
            r = pl.ds(rs * RS, RS)
            raw = _dot(xb_ref[r, :], wbf[...])
            if m == 2:
                v_out[r, cbs] = raw.astype(BF16)
            else:
                c = cos_ref[r, :]
                s = sin_ref[r, :]
                parts = []
                for hh in range(CB // DK):
                    x1 = raw[:, hh * DK:hh * DK + DK // 2]
                    x2 = raw[:, hh * DK + DK // 2:(hh + 1) * DK]
                    parts.append(x1 * c - x2 * s)
                    parts.append(x2 * c + x1 * s)
                outs[m][r, cbs] = jnp.concatenate(parts, axis=-1).astype(BF16)

    _stream_mats([wsq_hbm, wsk_hbm, wsv_hbm], wbuf, wbf, sems, consume)


def _swa_proj(xb, wsq, wsk, wsv, cos, sin):
    vm = lambda: pl.BlockSpec(memory_space=pltpu.VMEM)
    hbm = lambda: pl.BlockSpec(memory_space=pl.ANY)
    out_sd = jax.ShapeDtypeStruct((T, H * DK), BF16)
    return pl.pallas_call(
        _mega_swa_body,
        in_specs=[vm(), hbm(), hbm(), hbm(), vm(), vm()],
        out_specs=[vm()] * 3,
        out_shape=[out_sd] * 3,
        scratch_shapes=[pltpu.VMEM((2, DM, CB), F32),
                        pltpu.VMEM((DM, CB), BF16),
                        pltpu.SemaphoreType.DMA((2,))],
        compiler_params=_compiler_params((), 48),
        name="swa_proj",
    )(xb, wsq, wsk, wsv, cos, sin)


GCH = 128


GH = 2


def _gla_body(q_ref, k_ref, v_ref, g_ref, roh_ref, rrep_ref, o_ref, s_ref):
    s_ref[...] = jnp.zeros_like(s_ref)
    ri = lax.broadcasted_iota(jnp.int32, (GCH, GCH), 0)
    ci = lax.broadcasted_iota(jnp.int32, (GCH, GCH), 1)
    ltri = jnp.where(ri >= ci, 1.0, 0.0).astype(F32)

    def chunk(c, carry):
        sl = pl.ds(pl.multiple_of(c * GCH, GCH), GCH)
        rc = roh_ref[sl, :]
        mm = _dot_t(rc, rc) * ltri
        rr = rrep_ref[sl, :]
        for j in range(GH):
            hs = slice(j * DK, (j + 1) * DK)
            gc = g_ref[sl, hs].astype(F32)
            G = _dot(ltri, gc)
            Gt = G[GCH - 1:GCH, :]
            kc = k_ref[sl, hs]
            qib = (q_ref[sl, hs] * jnp.exp(G)).astype(BF16)
            ke = (kc * jnp.exp(-G)).astype(BF16)
            kd = (kc * jnp.exp(Gt - G)).astype(BF16)
            A = _dot_t(qib, ke) * mm
            vc = v_ref[sl, hs]
            oc = _dot(A.astype(BF16), vc)
            qi_st = jnp.concatenate([qib] * P, axis=1) * rr
            kd_st = jnp.concatenate([kd] * P, axis=1) * rr
            egt_rep = jnp.concatenate([jnp.exp(Gt)] * P, axis=1)
            sj = s_ref[pl.ds(j * DV, DV), :]
            oc += _dot_t(qi_st, sj.astype(BF16))
            s_ref[pl.ds(j * DV, DV), :] = sj * egt_rep + _dot_0(vc, kd_st)
            o_ref[sl, hs] = oc
        return carry

    lax.fori_loop(0, T // GCH, chunk, 0)


def _gla(q, k, v, g, r_oh, r_rep):
    grid = (H // GH,)
    head = lambda h: (0, h)
    fixed = lambda h: (0, 0)
    return pl.pallas_call(
        _gla_body,
        grid=grid,
        in_specs=[pl.BlockSpec((T, GH * DK), head)] * 3 + [
            pl.BlockSpec((T, GH * DK), head),
            pl.BlockSpec((T, 128), fixed),
            pl.BlockSpec((T, P * DK), fixed)],
        out_specs=pl.BlockSpec((T, GH * DV), head),
        out_shape=jax.ShapeDtypeStruct((T, H * DV), F32),
        scratch_shapes=[pltpu.VMEM((GH * DV, P * DK), F32)],
        compiler_params=_compiler_params(("arbitrary",), 48),
        name="sse_gla",
    )(q, k, v, g, r_oh, r_rep)


def _swa_body(q_ref, kp_ref, vp_ref, o_ref):
    bq = q_ref.shape[0]
    span = bq + WINDOW
    qb = pl.program_id(1)
    sl = pl.ds(pl.multiple_of(qb * bq, bq), span)
    ksl = kp_ref[sl, :]
    s = _dot_t(q_ref[...], ksl) * (DK ** -0.5)
    tl = lax.broadcasted_iota(jnp.int32, (bq, span), 0)
    jj = lax.broadcasted_iota(jnp.int32, (bq, span), 1)
    allowed = (jj > tl) & (jj <= tl + WINDOW) & (jj + qb * bq >= WINDOW)
    s = jnp.where(allowed, s, -1e30)
    m = jnp.max(s, axis=-1, keepdims=True)
    e = jnp.exp(s - m)
    denom = jnp.sum(e, axis=-1, keepdims=True)
    oc = _dot(e.astype(BF16), vp_ref[sl, :])
    o_ref[...] = (oc / denom).astype(BF16)


def _swa(qs, ks_pad, vs_pad):
    bq = 1024
    grid = (H, T // bq)
    qmap = lambda h, i: (i, h)
    kvmap = lambda h, i: (0, h)
    return pl.pallas_call(
        _swa_body,
        grid=grid,
        in_specs=[pl.BlockSpec((bq, DK), qmap),
                  pl.BlockSpec((T + WINDOW, DK), kvmap),
                  pl.BlockSpec((T + WINDOW, DK), kvmap)],
        out_specs=pl.BlockSpec((bq, DK), qmap),
        out_shape=jax.ShapeDtypeStruct((T, H * DK), BF16),
        compiler_params=_compiler_params(("arbitrary", "arbitrary"), 50),
        name="swa_attn",
    )(qs, ks_pad, vs_pad)


def _rms(x, w):
    return x * lax.rsqrt(jnp.mean(x * x, axis=-1, keepdims=True) + EPS) * w


def _merge_body(o_ref, gate_ref, swa_ref, wso_ref, wwo_ref, wnorm_ref,
                wm1_ref, wm2_ref, out_ref):
    o = o_ref[...]
    wn = wnorm_ref[...]
    parts = []
    for h in range(H):
        parts.append(_rms(o[:, h * DV:(h + 1) * DV], wn))
    gated = (jnp.concatenate(parts, axis=-1) * gate_ref[...]).astype(BF16)
    s1 = _dot(gated, wso_ref[...])
    s2 = _dot(swa_ref[...], wwo_ref[...])
    out_ref[...] = _rms(s1, wm1_ref[...]) + _rms(s2, wm2_ref[...])


def _merge(o_sse, gate, swa, wso, wwo, wnorm, wm1, wm2):
    bt = 512
    grid = (T // bt,)
    row = lambda i: (i, 0)
    fixed = lambda i: (0, 0)
    wspec = lambda w: pl.BlockSpec(w.shape, fixed)
    return pl.pallas_call(
        _merge_body,
        grid=grid,
        in_specs=[pl.BlockSpec((bt, H * DV), row),
                  pl.BlockSpec((bt, H * DV), row),
                  pl.BlockSpec((bt, H * DK), row),
                  wspec(wso), wspec(wwo), wspec(wnorm), wspec(wm1),
                  wspec(wm2)],
        out_specs=pl.BlockSpec((bt, DM), row),
        out_shape=jax.ShapeDtypeStruct((T, DM), F32),
        compiler_params=_compiler_params(("arbitrary",), 48),
        name="merge_out",
    )(o_sse, gate, swa, wso, wwo, wnorm, wm1, wm2)


def kernel(x, params):
    p = params
    x2 = x[0]
    xb = x2.astype(BF16)

    e = x @ p['W_e']
    vals, idx = lax.top_k(e, 1)
    ws = jax.nn.softmax(vals, axis=-1)
    oh = jax.nn.one_hot(idx, P, dtype=e.dtype)
    w_route = jnp.einsum('btkp,btk->btp', oh, ws)[0]
    r_oh = jnp.pad(w_route, ((0, 0), (0, 128 - P))).astype(BF16)
    r_rep = jnp.repeat(w_route, DK, axis=1).astype(BF16)

    inv = 10000.0 ** (-jnp.arange(0, DK, 2, dtype=F32) / DK)
    fr = jnp.arange(T, dtype=F32)[:, None] * inv[None, :]
    cos, sin = jnp.cos(fr), jnp.sin(fr)

    smw = jnp.concatenate(
        [p['W_lq0'], p['W_lk0'], p['W_g0'],
         jnp.pad(p['W_gk0'], ((0, 0), (0, 112)))], axis=1).astype(BF16)
    wlq1 = p['W_lq1'].astype(BF16)
    wlk1 = p['W_lk1'].astype(BF16)
    wgk1p = jnp.pad(p['W_gk1'], ((0, 112), (0, 0))).astype(BF16)
    bgk = p['b_gk1'].reshape(1, -1)
    wg1 = p['W_g1'].astype(BF16)
    wso = p['W_sse_o'].astype(BF16)
    wwo = p['W_swa_o'].astype(BF16)

    q, k, v, g, gate = _sse_proj(xb, p['W_sse_q'], p['W_sse_k'], p['W_sse_v'],
                                 smw, wlq1, wlk1, wgk1p, wg1, bgk)
    qs, ks, vs = _swa_proj(xb, p['W_swa_q'], p['W_swa_k'], p['W_swa_v'],
                           cos, sin)
    o_sse = _gla(q, k, v, g, r_oh, r_rep)
    ks_pad = jnp.pad(ks, ((WINDOW, 0), (0, 0)))
    vs_pad = jnp.pad(vs, ((WINDOW, 0), (0, 0)))
    swa = _swa(qs, ks_pad, vs_pad)
    out = _merge(o_sse, gate, swa, wso, wwo,
                 p['w_onorm'].reshape(1, -1),
                 p['w_sse_merge'].reshape(1, -1),
                 p['w_swa_merge'].reshape(1, -1))
    return out.reshape(B, T, DM)
